# baseline (speedup 1.0000x reference)
import functools

import jax
import jax.numpy as jnp
from jax import lax
from jax.experimental import pallas as pl
from jax.experimental.pallas import tpu as pltpu
from jax.experimental.pallas import tpu_sc as plsc

_NC = 2
_NS = 16
_NW = _NC * _NS
_L = 16

_B = 1024
_H = 200
_D = 64
_ROWS = _B * _H
_RPW = _ROWS // _NW
_CHUNK = 128
_NCHUNK = _RPW // _CHUNK


def _make_kernel():
    mesh = plsc.VectorSubcoreMesh(core_axis_name="c", subcore_axis_name="s")

    @functools.partial(
        pl.kernel,
        out_type=jax.ShapeDtypeStruct((_ROWS, _D), jnp.float32),
        mesh=mesh,
        scratch_types=[
            pltpu.VMEM((_RPW,), jnp.int32),
            pltpu.VMEM((_CHUNK, _D), jnp.float32),
            pltpu.VMEM((_D,), jnp.float32),
            pltpu.SemaphoreType.DMA,
        ],
        compiler_params=pltpu.CompilerParams(use_tc_tiling_on_sc=False),
    )
    def emb_kernel(x_hbm, table_hbm, pe_hbm, out_hbm, idx_v, rows_v, pe_v, sem):
        wid = lax.axis_index("s") * _NC + lax.axis_index("c")
        base = wid * _RPW

        pltpu.sync_copy(x_hbm.at[pl.ds(base, _RPW)], idx_v)
        pltpu.sync_copy(pe_hbm, pe_v)
        pe_vecs = [pe_v[pl.ds(k * _L, _L)] for k in range(_D // _L)]

        @pl.loop(0, _NCHUNK)
        def _chunks(j):
            cbase = j * _CHUNK
            pltpu.async_copy(
                table_hbm.at[idx_v.at[pl.ds(cbase, _CHUNK)]], rows_v, sem
            ).wait()

            @pl.loop(0, _CHUNK)
            def _rows(r):
                for k in range(_D // _L):
                    sl = pl.ds(k * _L, _L)
                    rows_v[r, sl] = rows_v[r, sl] + pe_vecs[k]

            pltpu.sync_copy(rows_v, out_hbm.at[pl.ds(base + cbase, _CHUNK)])

    return emb_kernel


_emb_kernel = _make_kernel()


def kernel(x, table, pe):
    x_flat = x.reshape(_ROWS)
    pe_row = pe[x.shape[0]]
    out = _emb_kernel(x_flat, table, pe_row)
    return out.reshape(_B, _H, _D)


# speedup vs baseline: 1.0817x; 1.0817x over previous
import functools

import jax
import jax.numpy as jnp
from jax import lax
from jax.experimental import pallas as pl
from jax.experimental.pallas import tpu as pltpu
from jax.experimental.pallas import tpu_sc as plsc

_NC = 2
_NS = 16
_NW = _NC * _NS
_L = 16

_B = 1024
_H = 200
_D = 64
_ROWS = _B * _H
_RPW = _ROWS // _NW
_CHUNK = 128
_NCHUNK = _RPW // _CHUNK
_NBUF = 5
_NGRP = _NCHUNK // _NBUF


def _make_kernel():
    mesh = plsc.VectorSubcoreMesh(core_axis_name="c", subcore_axis_name="s")

    @functools.partial(
        pl.kernel,
        out_type=jax.ShapeDtypeStruct((_ROWS, _D), jnp.float32),
        mesh=mesh,
        scratch_types=[
            pltpu.VMEM((_RPW,), jnp.int32),
            [pltpu.VMEM((_CHUNK, _D), jnp.float32) for _ in range(_NBUF)],
            pltpu.VMEM((_D,), jnp.float32),
            pltpu.SemaphoreType.DMA((_NBUF,)),
            pltpu.SemaphoreType.DMA((_NBUF,)),
        ],
        compiler_params=pltpu.CompilerParams(use_tc_tiling_on_sc=False),
    )
    def emb_kernel(x_hbm, table_hbm, pe_hbm, out_hbm,
                   idx_v, bufs, pe_v, gsem, ssem):
        wid = lax.axis_index("s") * _NC + lax.axis_index("c")
        base = wid * _RPW

        pltpu.sync_copy(x_hbm.at[pl.ds(base, _RPW)], idx_v)
        pltpu.sync_copy(pe_hbm, pe_v)
        pe_vecs = [pe_v[pl.ds(k * _L, _L)] for k in range(_D // _L)]

        def gather_start(j, b):
            pltpu.async_copy(
                table_hbm.at[idx_v.at[pl.ds(j * _CHUNK, _CHUNK)]],
                bufs[b], gsem.at[b])

        def gather_wait(b):
            pltpu.make_async_copy(
                table_hbm.at[idx_v.at[pl.ds(0, _CHUNK)]],
                bufs[b], gsem.at[b]).wait()

        def store_start(j, b):
            pltpu.async_copy(
                bufs[b], out_hbm.at[pl.ds(base + j * _CHUNK, _CHUNK)],
                ssem.at[b])

        def store_wait(b):
            pltpu.make_async_copy(
                bufs[b], out_hbm.at[pl.ds(base, _CHUNK)], ssem.at[b]).wait()

        for b in range(_NBUF):
            gather_start(b, b)

        @pl.loop(0, _NGRP)
        def _grp(g):
            jbase = g * _NBUF
            for b in range(_NBUF):
                gather_wait(b)
                buf = bufs[b]

                @plsc.parallel_loop(0, _CHUNK, 1, unroll=4)
                def _rows(r):
                    for k in range(_D // _L):
                        sl = pl.ds(k * _L, _L)
                        buf[r, sl] = buf[r, sl] + pe_vecs[k]

                store_start(jbase + b, b)

                @pl.when(g < _NGRP - 1)
                def _next():
                    store_wait(b)
                    gather_start(jbase + _NBUF + b, b)

        for b in range(_NBUF):
            store_wait(b)

    return emb_kernel


_emb_kernel = _make_kernel()


def kernel(x, table, pe):
    x_flat = x.reshape(_ROWS)
    pe_row = pe[x.shape[0]]
    out = _emb_kernel(x_flat, table, pe_row)
    return out.reshape(_B, _H, _D)
